# baseline (speedup 1.0000x reference)
import jax
import jax.numpy as jnp
import numpy as np
from jax.experimental import pallas as pl
from jax.experimental.pallas import tpu as pltpu

B = 8
NPER = 1024
M1 = 512
M2 = 128
K = 64
R1 = 0.2
R2 = 0.4
N_ACTIONS = 12


def _argmax_lanes(v):
    n = v.shape[1]
    mx = jnp.max(v, axis=1, keepdims=True)
    iota = jax.lax.broadcasted_iota(jnp.int32, v.shape, 1)
    return jnp.min(jnp.where(v == mx, iota, n), axis=1).astype(jnp.int32)


def _onehot_pick(v, nxt):
    iota = jax.lax.broadcasted_iota(jnp.int32, v.shape, 1)
    return jnp.sum(jnp.where(iota == nxt[:, None], v, 0.0), axis=1, keepdims=True)


def _fps_loop(xs, ys, zs, m):
    x0 = xs[:, 0:1]
    y0 = ys[:, 0:1]
    z0 = zs[:, 0:1]
    d0 = (xs - x0) * (xs - x0) + (ys - y0) * (ys - y0) + (zs - z0) * (zs - z0)
    oiota = jax.lax.broadcasted_iota(jnp.int32, (B, m), 1)
    q0 = jnp.zeros((B, m), jnp.int32)
    px0 = jnp.broadcast_to(x0, (B, m))
    py0 = jnp.broadcast_to(y0, (B, m))
    pz0 = jnp.broadcast_to(z0, (B, m))

    def body(i, state):
        mind, q, pxs, pys, pzs = state
        nxt = _argmax_lanes(mind)
        px = _onehot_pick(xs, nxt)
        py = _onehot_pick(ys, nxt)
        pz = _onehot_pick(zs, nxt)
        sel = oiota == i
        q = jnp.where(sel, nxt[:, None], q)
        pxs = jnp.where(sel, px, pxs)
        pys = jnp.where(sel, py, pys)
        pzs = jnp.where(sel, pz, pzs)
        d = (xs - px) * (xs - px) + (ys - py) * (ys - py) + (zs - pz) * (zs - pz)
        return jnp.minimum(mind, d), q, pxs, pys, pzs

    _, q, pxs, pys, pzs = jax.lax.fori_loop(1, m, body, (d0, q0, px0, py0, pz0))
    return q, pxs, pys, pzs


def _fps_kernel(xs_ref, ys_ref, zs_ref,
                q1_ref, p1x_ref, p1y_ref, p1z_ref,
                q2_ref, p2x_ref, p2y_ref, p2z_ref):
    xs = xs_ref[...]
    ys = ys_ref[...]
    zs = zs_ref[...]
    q1, x1, y1, z1 = _fps_loop(xs, ys, zs, M1)
    q1_ref[...] = q1
    p1x_ref[...] = x1
    p1y_ref[...] = y1
    p1z_ref[...] = z1
    q2, x2, y2, z2 = _fps_loop(x1, y1, z1, M2)
    q2_ref[...] = q2
    p2x_ref[...] = x2
    p2y_ref[...] = y2
    p2z_ref[...] = z2


def _fps_pallas(xs, ys, zs):
    outs = pl.pallas_call(
        _fps_kernel,
        out_shape=(
            jax.ShapeDtypeStruct((B, M1), jnp.int32),
            jax.ShapeDtypeStruct((B, M1), jnp.float32),
            jax.ShapeDtypeStruct((B, M1), jnp.float32),
            jax.ShapeDtypeStruct((B, M1), jnp.float32),
            jax.ShapeDtypeStruct((B, M2), jnp.int32),
            jax.ShapeDtypeStruct((B, M2), jnp.float32),
            jax.ShapeDtypeStruct((B, M2), jnp.float32),
            jax.ShapeDtypeStruct((B, M2), jnp.float32),
        ),
    )(xs, ys, zs)
    return outs


def _cumsum_lanes(x):
    m, n = x.shape
    s = 1
    while s < n:
        shifted = jnp.concatenate(
            [jnp.zeros((m, s), x.dtype), x[:, : n - s]], axis=1)
        x = x + shifted
        s *= 2
    return x


def _sel_kernel(m, n, k, hi0, qxt_ref, qyt_ref, qzt_ref, xs_ref, ys_ref, zs_ref,
                dest_ref, mask_ref):
    b = pl.program_id(0)
    biota = jax.lax.broadcasted_iota(jnp.int32, (m, 8), 1)
    bsel = biota == b

    def col(ref):
        return jnp.sum(jnp.where(bsel, ref[...], 0.0), axis=1, keepdims=True)

    qx = col(qxt_ref)
    qy = col(qyt_ref)
    qz = col(qzt_ref)
    xs = xs_ref[0]
    ys = ys_ref[0]
    zs = zs_ref[0]
    d2 = (qx - xs) * (qx - xs) + (qy - ys) * (qy - ys) + (qz - zs) * (qz - zs)
    d2b = jax.lax.bitcast_convert_type(d2, jnp.int32)

    lo0 = jnp.zeros((m, 1), jnp.int32)
    hi_init = jnp.full((m, 1), hi0, jnp.int32)

    def bs_body(_, state):
        lo, hi = state
        active = lo < hi
        mid = lo + jax.lax.shift_right_logical(hi - lo, 1)
        cnt = jnp.sum(jnp.where(d2b <= mid, 1, 0), axis=1, keepdims=True)
        ge = cnt >= k
        nlo = jnp.where(ge, lo, mid + 1)
        nhi = jnp.where(ge, mid, hi)
        lo = jnp.where(active, nlo, lo)
        hi = jnp.where(active, nhi, hi)
        return lo, hi

    t, _ = jax.lax.fori_loop(0, 31, bs_body, (lo0, hi_init))

    strict = d2b < t
    tie = d2b == t
    si = jnp.where(strict, 1, 0)
    ti = jnp.where(tie, 1, 0)
    cs = _cumsum_lanes(si)
    ct = _cumsum_lanes(ti)
    ns = cs[:, n - 1 : n]
    nt = ct[:, n - 1 : n]
    dest = jnp.where(strict, cs - 1, jnp.where(tie, ns + ct - 1, -1))
    dest = jnp.where(dest < k, dest, -1)
    dest_ref[0] = dest
    nfill = jnp.minimum(ns + nt, k)
    kiota = jax.lax.broadcasted_iota(jnp.int32, (m, k), 1)
    mask_ref[0] = jnp.where(kiota < nfill, 1.0, 0.0)


def _sel_pallas(qxt, qyt, qzt, xs, ys, zs, r):
    import functools
    m = qxt.shape[0]
    n = xs.shape[1]
    hi0 = int(np.float32(r * r + 1e-12).view(np.int32))
    xs3 = xs.reshape(B, 1, n)
    ys3 = ys.reshape(B, 1, n)
    zs3 = zs.reshape(B, 1, n)
    dest, mask01 = pl.pallas_call(
        functools.partial(_sel_kernel, m, n, K, hi0),
        grid=(B,),
        in_specs=[
            pl.BlockSpec((m, 8), lambda b: (0, 0)),
            pl.BlockSpec((m, 8), lambda b: (0, 0)),
            pl.BlockSpec((m, 8), lambda b: (0, 0)),
            pl.BlockSpec((1, 1, n), lambda b: (b, 0, 0)),
            pl.BlockSpec((1, 1, n), lambda b: (b, 0, 0)),
            pl.BlockSpec((1, 1, n), lambda b: (b, 0, 0)),
        ],
        out_specs=[
            pl.BlockSpec((1, m, n), lambda b: (b, 0, 0)),
            pl.BlockSpec((1, m, K), lambda b: (b, 0, 0)),
        ],
        out_shape=[
            jax.ShapeDtypeStruct((B, m, n), jnp.int32),
            jax.ShapeDtypeStruct((B, m, K), jnp.float32),
        ],
    )(qxt, qyt, qzt, xs3, ys3, zs3)
    return dest, mask01


def _nbr_from_dest(dest):
    Bb, m, n = dest.shape
    ci = jnp.broadcast_to(jnp.arange(n, dtype=jnp.int32), (Bb, m, n))
    d = jnp.where(dest >= 0, dest, K)
    nbr = jnp.zeros((Bb, m, K), jnp.int32)
    bi = jnp.arange(Bb, dtype=jnp.int32)[:, None, None]
    qi = jnp.arange(m, dtype=jnp.int32)[None, :, None]
    return nbr.at[bi, qi, d].set(ci, mode="drop")


def _radius(pos_b, qpos, r):
    d2 = jnp.sum((qpos[:, :, None, :] - pos_b[:, None, :, :]) ** 2, axis=-1)
    negv, nbr = jax.lax.top_k(-d2, K)
    mask = (-negv) <= r * r + 1e-12
    return nbr, mask


def _gather(xb, nbr):
    Bb, m, k = nbr.shape
    out = jnp.take_along_axis(xb, nbr.reshape(Bb, m * k)[:, :, None], axis=1)
    return out.reshape(Bb, m, k, xb.shape[-1])


def _take(xb, idx):
    return jnp.take_along_axis(xb, idx[:, :, None], axis=1)


def _mlp_bn(x, layers, mask=None):
    for lyr in layers[:-1]:
        W, b, g, be = lyr
        x = x @ W + b
        if mask is None:
            mu = jnp.mean(x, axis=0)
            var = jnp.mean((x - mu) ** 2, axis=0)
        else:
            w = mask / jnp.maximum(jnp.sum(mask), 1.0)
            mu = jnp.sum(w[:, None] * x, axis=0)
            var = jnp.sum(w[:, None] * (x - mu) ** 2, axis=0)
        x = (x - mu) / jnp.sqrt(var + 1e-5) * g + be
        x = jax.nn.relu(x)
    W, b = layers[-1]
    return x @ W + b


def _point_conv(x_b, pos_b, qpos, nbr, mask, layers):
    h = _gather(pos_b, nbr) - qpos[:, :, None, :]
    if x_b is not None:
        h = jnp.concatenate([_gather(x_b, nbr), h], axis=-1)
    Bb, m, k, F = h.shape
    out = _mlp_bn(h.reshape(Bb * m * k, F), layers, mask.reshape(Bb * m * k).astype(jnp.float32))
    out = out.reshape(Bb, m, k, -1)
    out = jnp.where(mask[..., None], out, -jnp.inf)
    return jnp.max(out, axis=2)


def _tail_kernel(h_ref, w1, b1, g1, be1, w2, b2, g2, be2, w3, b3,
                 pw1, pb1, pw2, pb2, pw3, pb3,
                 vw1, vb1, vw2, vb2, vw3, vb3,
                 probs_ref, value_ref):
    x = h_ref[...]
    x = jnp.dot(x, w1[...], preferred_element_type=jnp.float32) + b1[...]
    mu = jnp.mean(x, axis=0)
    var = jnp.mean((x - mu) ** 2, axis=0)
    x = (x - mu) / jnp.sqrt(var + 1e-5) * g1[...] + be1[...]
    x = jax.nn.relu(x)
    x = jnp.dot(x, w2[...], preferred_element_type=jnp.float32) + b2[...]
    mu = jnp.mean(x, axis=0)
    var = jnp.mean((x - mu) ** 2, axis=0)
    x = (x - mu) / jnp.sqrt(var + 1e-5) * g2[...] + be2[...]
    x = jax.nn.relu(x)
    x = jnp.dot(x, w3[...], preferred_element_type=jnp.float32) + b3[...]
    g = jnp.max(x.reshape(B, M2, x.shape[-1]), axis=1)
    p = jax.nn.relu(jnp.dot(g, pw1[...], preferred_element_type=jnp.float32) + pb1[...])
    p = jax.nn.relu(jnp.dot(p, pw2[...], preferred_element_type=jnp.float32) + pb2[...])
    logits = jnp.dot(p, pw3[...], preferred_element_type=jnp.float32) + pb3[...]
    probs_ref[...] = jax.nn.softmax(logits, axis=-1)
    v = jax.nn.relu(jnp.dot(g, vw1[...], preferred_element_type=jnp.float32) + vb1[...])
    v = jax.nn.relu(jnp.dot(v, vw2[...], preferred_element_type=jnp.float32) + vb2[...])
    value_ref[...] = jnp.dot(v, vw3[...], preferred_element_type=jnp.float32) + vb3[...]


def _dense_tail(h, params):
    (w1, b1, g1, be1), (w2, b2, g2, be2), (w3, b3) = params["nn3"]
    (pw1, pb1), (pw2, pb2), (pw3, pb3) = params["pi"]
    (vw1, vb1), (vw2, vb2), (vw3, vb3) = params["value"]
    probs, value = pl.pallas_call(
        _tail_kernel,
        out_shape=(
            jax.ShapeDtypeStruct((B, N_ACTIONS), jnp.float32),
            jax.ShapeDtypeStruct((B, 1), jnp.float32),
        ),
    )(h, w1, b1, g1, be1, w2, b2, g2, be2, w3, b3,
      pw1, pb1, pw2, pb2, pw3, pb3,
      vw1, vb1, vw2, vb2, vw3, vb3)
    return probs, value[:, 0]


def kernel(pos, ptr, params):
    Bb = ptr.shape[0] - 1
    n = pos.shape[0] // Bb
    pos_b = pos.reshape(Bb, n, 3)
    pos_b = jax.lax.stop_gradient(pos_b)
    xs = pos_b[:, :, 0]
    ys = pos_b[:, :, 1]
    zs = pos_b[:, :, 2]
    (q1, p1x, p1y, p1z, q2, p2x, p2y, p2z) = _fps_pallas(xs, ys, zs)
    pos1 = jnp.stack([p1x, p1y, p1z], axis=-1)
    pos2 = jnp.stack([p2x, p2y, p2z], axis=-1)
    dest1, mask1f = _sel_pallas(p1x.T, p1y.T, p1z.T, xs, ys, zs, R1)
    nbr1 = _nbr_from_dest(dest1)
    mask1 = mask1f > 0.5
    dest2, mask2f = _sel_pallas(p2x.T, p2y.T, p2z.T, p1x, p1y, p1z, R2)
    nbr2 = _nbr_from_dest(dest2)
    mask2 = mask2f > 0.5

    x1 = _point_conv(None, pos_b, pos1, nbr1, mask1, params["nn1"])
    x2 = _point_conv(x1, pos1, pos2, nbr2, mask2, params["nn2"])
    h = jnp.concatenate([x2, pos2], axis=-1).reshape(Bb * M2, -1)
    return _dense_tail(h, params)


# speedup vs baseline: 15.6008x; 15.6008x over previous
import jax
import jax.numpy as jnp
import numpy as np
from jax.experimental import pallas as pl
from jax.experimental.pallas import tpu as pltpu

B = 8
NPER = 1024
M1 = 512
M2 = 128
K = 64
R1 = 0.2
R2 = 0.4
N_ACTIONS = 12


def _argmax_lanes(v):
    n = v.shape[1]
    mx = jnp.max(v, axis=1, keepdims=True)
    iota = jax.lax.broadcasted_iota(jnp.int32, v.shape, 1)
    return jnp.min(jnp.where(v == mx, iota, n), axis=1).astype(jnp.int32)


def _onehot_pick(v, nxt):
    iota = jax.lax.broadcasted_iota(jnp.int32, v.shape, 1)
    return jnp.sum(jnp.where(iota == nxt[:, None], v, 0.0), axis=1, keepdims=True)


def _fps_loop(xs, ys, zs, m):
    x0 = xs[:, 0:1]
    y0 = ys[:, 0:1]
    z0 = zs[:, 0:1]
    d0 = (xs - x0) * (xs - x0) + (ys - y0) * (ys - y0) + (zs - z0) * (zs - z0)
    oiota = jax.lax.broadcasted_iota(jnp.int32, (B, m), 1)
    q0 = jnp.zeros((B, m), jnp.int32)
    px0 = jnp.broadcast_to(x0, (B, m))
    py0 = jnp.broadcast_to(y0, (B, m))
    pz0 = jnp.broadcast_to(z0, (B, m))

    def body(i, state):
        mind, q, pxs, pys, pzs = state
        nxt = _argmax_lanes(mind)
        px = _onehot_pick(xs, nxt)
        py = _onehot_pick(ys, nxt)
        pz = _onehot_pick(zs, nxt)
        sel = oiota == i
        q = jnp.where(sel, nxt[:, None], q)
        pxs = jnp.where(sel, px, pxs)
        pys = jnp.where(sel, py, pys)
        pzs = jnp.where(sel, pz, pzs)
        d = (xs - px) * (xs - px) + (ys - py) * (ys - py) + (zs - pz) * (zs - pz)
        return jnp.minimum(mind, d), q, pxs, pys, pzs

    _, q, pxs, pys, pzs = jax.lax.fori_loop(1, m, body, (d0, q0, px0, py0, pz0))
    return q, pxs, pys, pzs


def _fps_kernel(xs_ref, ys_ref, zs_ref,
                q1_ref, p1x_ref, p1y_ref, p1z_ref,
                q2_ref, p2x_ref, p2y_ref, p2z_ref):
    xs = xs_ref[...]
    ys = ys_ref[...]
    zs = zs_ref[...]
    q1, x1, y1, z1 = _fps_loop(xs, ys, zs, M1)
    q1_ref[...] = q1
    p1x_ref[...] = x1
    p1y_ref[...] = y1
    p1z_ref[...] = z1
    q2, x2, y2, z2 = _fps_loop(x1, y1, z1, M2)
    q2_ref[...] = q2
    p2x_ref[...] = x2
    p2y_ref[...] = y2
    p2z_ref[...] = z2


def _fps_pallas(xs, ys, zs):
    outs = pl.pallas_call(
        _fps_kernel,
        out_shape=(
            jax.ShapeDtypeStruct((B, M1), jnp.int32),
            jax.ShapeDtypeStruct((B, M1), jnp.float32),
            jax.ShapeDtypeStruct((B, M1), jnp.float32),
            jax.ShapeDtypeStruct((B, M1), jnp.float32),
            jax.ShapeDtypeStruct((B, M2), jnp.int32),
            jax.ShapeDtypeStruct((B, M2), jnp.float32),
            jax.ShapeDtypeStruct((B, M2), jnp.float32),
            jax.ShapeDtypeStruct((B, M2), jnp.float32),
        ),
    )(xs, ys, zs)
    return outs


def _cumsum_lanes(x):
    m, n = x.shape
    s = 1
    while s < n:
        shifted = jnp.concatenate(
            [jnp.zeros((m, s), x.dtype), x[:, : n - s]], axis=1)
        x = x + shifted
        s *= 2
    return x


def _sel_kernel(m, n, k, hi0, qxt_ref, qyt_ref, qzt_ref, xs_ref, ys_ref, zs_ref,
                dest_ref, mask_ref):
    b = pl.program_id(0)
    biota = jax.lax.broadcasted_iota(jnp.int32, (m, 8), 1)
    bsel = biota == b

    def col(ref):
        return jnp.sum(jnp.where(bsel, ref[...], 0.0), axis=1, keepdims=True)

    qx = col(qxt_ref)
    qy = col(qyt_ref)
    qz = col(qzt_ref)
    xs = xs_ref[0]
    ys = ys_ref[0]
    zs = zs_ref[0]
    d2 = (qx - xs) * (qx - xs) + (qy - ys) * (qy - ys) + (qz - zs) * (qz - zs)
    d2b = jax.lax.bitcast_convert_type(d2, jnp.int32)

    lo0 = jnp.zeros((m, 1), jnp.int32)
    hi_init = jnp.full((m, 1), hi0, jnp.int32)

    def bs_body(_, state):
        lo, hi = state
        active = lo < hi
        mid = lo + jax.lax.shift_right_logical(hi - lo, 1)
        cnt = jnp.sum(jnp.where(d2b <= mid, 1, 0), axis=1, keepdims=True)
        ge = cnt >= k
        nlo = jnp.where(ge, lo, mid + 1)
        nhi = jnp.where(ge, mid, hi)
        lo = jnp.where(active, nlo, lo)
        hi = jnp.where(active, nhi, hi)
        return lo, hi

    t, _ = jax.lax.fori_loop(0, 31, bs_body, (lo0, hi_init))

    strict = d2b < t
    tie = d2b == t
    si = jnp.where(strict, 1, 0)
    ti = jnp.where(tie, 1, 0)
    cs = _cumsum_lanes(si)
    ct = _cumsum_lanes(ti)
    ns = cs[:, n - 1 : n]
    nt = ct[:, n - 1 : n]
    dest = jnp.where(strict, cs - 1, jnp.where(tie, ns + ct - 1, -1))
    dest = jnp.where(dest < k, dest, -1)
    dest_ref[0] = dest
    nfill = jnp.minimum(ns + nt, k)
    kiota = jax.lax.broadcasted_iota(jnp.int32, (m, k), 1)
    mask_ref[0] = jnp.where(kiota < nfill, 1.0, 0.0)


def _sel_pallas(qxt, qyt, qzt, xs, ys, zs, r):
    import functools
    m = qxt.shape[0]
    n = xs.shape[1]
    hi0 = int(np.float32(r * r + 1e-12).view(np.int32))
    xs3 = xs.reshape(B, 1, n)
    ys3 = ys.reshape(B, 1, n)
    zs3 = zs.reshape(B, 1, n)
    dest, mask01 = pl.pallas_call(
        functools.partial(_sel_kernel, m, n, K, hi0),
        grid=(B,),
        in_specs=[
            pl.BlockSpec((m, 8), lambda b: (0, 0)),
            pl.BlockSpec((m, 8), lambda b: (0, 0)),
            pl.BlockSpec((m, 8), lambda b: (0, 0)),
            pl.BlockSpec((1, 1, n), lambda b: (b, 0, 0)),
            pl.BlockSpec((1, 1, n), lambda b: (b, 0, 0)),
            pl.BlockSpec((1, 1, n), lambda b: (b, 0, 0)),
        ],
        out_specs=[
            pl.BlockSpec((1, m, n), lambda b: (b, 0, 0)),
            pl.BlockSpec((1, m, K), lambda b: (b, 0, 0)),
        ],
        out_shape=[
            jax.ShapeDtypeStruct((B, m, n), jnp.int32),
            jax.ShapeDtypeStruct((B, m, K), jnp.float32),
        ],
    )(qxt, qyt, qzt, xs3, ys3, zs3)
    return dest, mask01


def _l1(h, xg, wb, bb, wa):
    x = jnp.dot(h, wb[...], preferred_element_type=jnp.float32) + bb[...]
    if xg is not None:
        x = x + jnp.dot(xg[...], wa[...], preferred_element_type=jnp.float32)
    return x


def _bnorm(x, mu, var, g, be):
    return jax.nn.relu((x - mu) / jnp.sqrt(var + 1e-5) * g[...] + be[...])


def _conv_passA(nblk, ntot, has_xg, *refs):
    if has_xg:
        (h_ref, xg_ref, mask_ref, wb, bb, wa, stats_ref, acc) = refs
        xg = xg_ref[...]
    else:
        (h_ref, mask_ref, wb, bb, stats_ref, acc) = refs
        xg = None
        wa = None
    i = pl.program_id(0)
    x = _l1(h_ref[...], xg, wb, bb, wa)
    cm = jnp.sum(mask_ref[...])
    s1 = jnp.sum(x, axis=0, keepdims=True)
    s2 = jnp.sum(x * x, axis=0, keepdims=True)
    sc = jnp.full((1, x.shape[1]), cm, jnp.float32)

    @pl.when(i == 0)
    def _():
        acc[...] = jnp.zeros_like(acc)

    for r in range(8):
        sel = (i % 8) == r
        acc[r : r + 1, :] += jnp.where(sel, s1, 0.0)
        acc[8 + r : 9 + r, :] += jnp.where(sel, s2, 0.0)
        acc[16 + r : 17 + r, :] += jnp.where(sel, sc, 0.0)

    @pl.when(i == nblk - 1)
    def _():
        cnt = jnp.sum(acc[16:24, :], axis=0, keepdims=True)
        n = jnp.maximum(cnt, 1.0)
        nmiss = float(ntot) - cnt
        cr = bb[...].reshape(1, -1)
        swx = jnp.sum(acc[0:8, :], axis=0, keepdims=True) - nmiss * cr
        swx2 = jnp.sum(acc[8:16, :], axis=0, keepdims=True) - nmiss * (cr * cr)
        mu = swx / n
        var = swx2 / n - mu * mu
        stats_ref[0:1, :] = mu
        stats_ref[1:2, :] = var
        stats_ref[2:3, :] = cnt


def _conv_passB(nblk, ntot, has_xg, *refs):
    if has_xg:
        (h_ref, xg_ref, wb, bb, wa, st1, g1, be1, w2, b2, stats_ref, acc) = refs
        xg = xg_ref[...]
    else:
        (h_ref, wb, bb, st1, g1, be1, w2, b2, stats_ref, acc) = refs
        xg = None
        wa = None
    i = pl.program_id(0)
    x = _l1(h_ref[...], xg, wb, bb, wa)
    mu1 = st1[0:1, :]
    var1 = st1[1:2, :]
    x = _bnorm(x, mu1, var1, g1, be1)
    x = jnp.dot(x, w2[...], preferred_element_type=jnp.float32) + b2[...]
    s1 = jnp.sum(x, axis=0, keepdims=True)
    s2 = jnp.sum(x * x, axis=0, keepdims=True)

    @pl.when(i == 0)
    def _():
        acc[...] = jnp.zeros_like(acc)

    for r in range(8):
        sel = (i % 8) == r
        acc[r : r + 1, :] += jnp.where(sel, s1, 0.0)
        acc[8 + r : 9 + r, :] += jnp.where(sel, s2, 0.0)

    @pl.when(i == nblk - 1)
    def _():
        nvec = st1[2:3, :]
        n1 = jnp.maximum(nvec[0:1, 0:1], 1.0)
        nmiss = float(ntot) - nvec[0:1, 0:1]
        cr1 = _bnorm(bb[...].reshape(1, -1), mu1, var1, g1, be1)
        cr = jnp.dot(cr1, w2[...], preferred_element_type=jnp.float32) + b2[...]
        swx = jnp.sum(acc[0:8, :], axis=0, keepdims=True) - nmiss * cr
        swx2 = jnp.sum(acc[8:16, :], axis=0, keepdims=True) - nmiss * (cr * cr)
        mu = swx / n1
        var = swx2 / n1 - mu * mu
        stats_ref[0:1, :] = mu
        stats_ref[1:2, :] = var


def _conv_passC(has_xg, *refs):
    if has_xg:
        (h_ref, xg_ref, mask_ref, wb, bb, wa, st1, g1, be1, w2, b2,
         st2, g2, be2, w3, b3, out_ref) = refs
        xg = xg_ref[...]
    else:
        (h_ref, mask_ref, wb, bb, st1, g1, be1, w2, b2,
         st2, g2, be2, w3, b3, out_ref) = refs
        xg = None
        wa = None
    x = _l1(h_ref[...], xg, wb, bb, wa)
    x = _bnorm(x, st1[0:1, :], st1[1:2, :], g1, be1)
    x = jnp.dot(x, w2[...], preferred_element_type=jnp.float32) + b2[...]
    x = _bnorm(x, st2[0:1, :], st2[1:2, :], g2, be2)
    x = jnp.dot(x, w3[...], preferred_element_type=jnp.float32) + b3[...]
    nq = mask_ref.shape[0]
    x3 = x.reshape(nq, K, x.shape[1])
    mb = mask_ref[...][:, :, None] > 0.5
    x3 = jnp.where(mb, x3, -jnp.inf)
    out_ref[...] = jnp.max(x3, axis=1)


def _pointconv_pallas(hmat, xg, mask01, layers):
    ntot = hmat.shape[0]
    blk = 8192
    nblk = ntot // blk
    nqb = blk // K
    (w1, b1, g1, be1), (w2, b2, g2, be2), (w3, b3) = layers
    has_xg = xg is not None
    if has_xg:
        wa, wb = w1[: xg.shape[1]], w1[xg.shape[1]:]
    else:
        wa, wb = None, w1
    c1 = w1.shape[1]
    c2 = w2.shape[1]
    c3 = w3.shape[1]
    import functools
    hspec = pl.BlockSpec((blk, hmat.shape[1]), lambda i: (i, 0))
    xgspec = pl.BlockSpec((blk, xg.shape[1]), lambda i: (i, 0)) if has_xg else None
    mspec = pl.BlockSpec((nqb, K), lambda i: (i, 0))
    full = lambda a: pl.BlockSpec(a.shape, lambda i: (0, 0) if a.ndim == 2 else (0,))

    ins = [hmat] + ([xg] if has_xg else []) + [mask01, wb, b1] + ([wa] if has_xg else [])
    specs = [hspec] + ([xgspec] if has_xg else []) + [mspec, full(wb), full(b1)] + ([full(wa)] if has_xg else [])
    st1 = pl.pallas_call(
        functools.partial(_conv_passA, nblk, ntot, has_xg),
        grid=(nblk,),
        in_specs=specs,
        out_specs=pl.BlockSpec((8, c1), lambda i: (0, 0)),
        out_shape=jax.ShapeDtypeStruct((8, c1), jnp.float32),
        scratch_shapes=[pltpu.VMEM((24, c1), jnp.float32)],
    )(*ins)

    ins = [hmat] + ([xg] if has_xg else []) + [wb, b1] + ([wa] if has_xg else []) + [st1, g1, be1, w2, b2]
    specs = [hspec] + ([xgspec] if has_xg else []) + [full(wb), full(b1)] + ([full(wa)] if has_xg else []) \
        + [full(st1), full(g1), full(be1), full(w2), full(b2)]
    st2 = pl.pallas_call(
        functools.partial(_conv_passB, nblk, ntot, has_xg),
        grid=(nblk,),
        in_specs=specs,
        out_specs=pl.BlockSpec((8, c2), lambda i: (0, 0)),
        out_shape=jax.ShapeDtypeStruct((8, c2), jnp.float32),
        scratch_shapes=[pltpu.VMEM((16, c2), jnp.float32)],
    )(*ins)

    ins = [hmat] + ([xg] if has_xg else []) + [mask01, wb, b1] + ([wa] if has_xg else []) \
        + [st1, g1, be1, w2, b2, st2, g2, be2, w3, b3]
    specs = [hspec] + ([xgspec] if has_xg else []) + [mspec, full(wb), full(b1)] + ([full(wa)] if has_xg else []) \
        + [full(st1), full(g1), full(be1), full(w2), full(b2),
           full(st2), full(g2), full(be2), full(w3), full(b3)]
    out = pl.pallas_call(
        functools.partial(_conv_passC, has_xg),
        grid=(nblk,),
        in_specs=specs,
        out_specs=pl.BlockSpec((nqb, c3), lambda i: (i, 0)),
        out_shape=jax.ShapeDtypeStruct((ntot // K, c3), jnp.float32),
    )(*ins)
    return out


from jax.experimental.pallas import tpu_sc as plsc
import functools


def _sc_compact_kernel(nq, ncand, qper, grp, emit_nbr, dest_ref, tx_ref, ty_ref,
                       tz_ref, qx_ref, qy_ref, qz_ref, *rest):
    if emit_nbr:
        (hout_ref, nout_ref, tabx, taby, tabz, qxv, qyv, qzv, dbuf, hbuf, nbuf_v,
         sem) = rest
    else:
        (hout_ref, tabx, taby, tabz, qxv, qyv, qzv, dbuf, hbuf, sem) = rest
        nout_ref = None
        nbuf_v = None
    info = plsc.get_sparse_core_info()
    nc = info.num_cores
    wid = jax.lax.axis_index("s") * nc + jax.lax.axis_index("c")
    qbase = wid * qper
    bat = qbase // (nq // B)
    cbase = bat * ncand
    pltpu.sync_copy(tx_ref.at[pl.ds(cbase, ncand)], tabx)
    pltpu.sync_copy(ty_ref.at[pl.ds(cbase, ncand)], taby)
    pltpu.sync_copy(tz_ref.at[pl.ds(cbase, ncand)], tabz)
    pltpu.sync_copy(qx_ref.at[pl.ds(qbase, qper)], qxv)
    pltpu.sync_copy(qy_ref.at[pl.ds(qbase, qper)], qyv)
    pltpu.sync_copy(qz_ref.at[pl.ds(qbase, qper)], qzv)
    ngrp = qper // grp
    iota16 = jax.lax.broadcasted_iota(jnp.int32, (16,), 0)
    zero16 = jnp.zeros((16,), jnp.float32)

    def grp_body(g, carry):
        pltpu.sync_copy(dest_ref.at[pl.ds((qbase + g * grp) * ncand, grp * ncand)],
                        dbuf)
        for qq in range(grp):
            q = g * grp + qq
            obase = q * K * 3
            for r in range(K * 3 // 16):
                hbuf[pl.ds(obase + 16 * r, 16)] = zero16
            if emit_nbr:
                for r in range(K // 16):
                    nbuf_v[pl.ds(q * K + 16 * r, 16)] = jnp.full((16,), NPER * 4, jnp.int32)
            qidx = jnp.full((16,), q, jnp.int32)
            qxs = plsc.load_gather(qxv, [qidx])
            qys = plsc.load_gather(qyv, [qidx])
            qzs = plsc.load_gather(qzv, [qidx])
            for j in range(ncand // 16):
                slots = dbuf[pl.ds(qq * ncand + 16 * j, 16)]
                msk = slots >= 0
                addr = jnp.full((16,), obase, jnp.int32) + slots * 3
                vx = tabx[pl.ds(16 * j, 16)] - qxs
                vy = taby[pl.ds(16 * j, 16)] - qys
                vz = tabz[pl.ds(16 * j, 16)] - qzs
                plsc.store_scatter(hbuf, [addr], vx, mask=msk)
                plsc.store_scatter(hbuf, [addr + 1], vy, mask=msk)
                plsc.store_scatter(hbuf, [addr + 2], vz, mask=msk)
                if emit_nbr:
                    gval = jnp.full((16,), bat * ncand + 16 * j, jnp.int32) + iota16
                    plsc.store_scatter(nbuf_v, [jnp.full((16,), q * K, jnp.int32) + slots],
                                       gval, mask=msk)
        return carry

    jax.lax.fori_loop(0, ngrp, grp_body, 0)
    pltpu.sync_copy(hbuf, hout_ref.at[pl.ds(qbase * K * 3, qper * K * 3)])
    if emit_nbr:
        pltpu.sync_copy(nbuf_v, nout_ref.at[pl.ds(qbase * K, qper * K)])


def _sc_compact(dest, tabx, taby, tabz, qx, qy, qz, emit_nbr):
    nq, ncand = dest.shape
    qper = nq // 32
    grp = max(1, min(8, 8192 // ncand))
    mesh = plsc.VectorSubcoreMesh(core_axis_name="c", subcore_axis_name="s")
    out_type = [jax.ShapeDtypeStruct((nq * K * 3,), jnp.float32)]
    if emit_nbr:
        out_type.append(jax.ShapeDtypeStruct((nq * K,), jnp.int32))
    scratch = [
        pltpu.VMEM((ncand,), jnp.float32),
        pltpu.VMEM((ncand,), jnp.float32),
        pltpu.VMEM((ncand,), jnp.float32),
        pltpu.VMEM((qper,), jnp.float32),
        pltpu.VMEM((qper,), jnp.float32),
        pltpu.VMEM((qper,), jnp.float32),
        pltpu.VMEM((grp * ncand,), jnp.int32),
        pltpu.VMEM((qper * K * 3,), jnp.float32),
    ]
    if emit_nbr:
        scratch.append(pltpu.VMEM((qper * K,), jnp.int32))
    scratch.append(pltpu.SemaphoreType.DMA)
    kfn = pl.kernel(
        functools.partial(_sc_compact_kernel, nq, ncand, qper, grp, emit_nbr),
        mesh=mesh,
        out_type=tuple(out_type) if emit_nbr else out_type[0],
        scratch_types=scratch,
        compiler_params=pltpu.CompilerParams(needs_layout_passes=False),
    )
    return kfn(dest.reshape(-1), tabx.reshape(-1), taby.reshape(-1),
               tabz.reshape(-1), qx.reshape(-1), qy.reshape(-1), qz.reshape(-1))


def _sc_rowgather_kernel(nrow, dcol, idx_ref, tab_ref, out_ref, idxv, rows, sem):
    info = plsc.get_sparse_core_info()
    nc = info.num_cores
    wid = jax.lax.axis_index("s") * nc + jax.lax.axis_index("c")
    per = nrow // 32
    base = wid * per

    def body(c, carry):
        off = base + c * 128
        pltpu.sync_copy(idx_ref.at[pl.ds(off, 128)], idxv)
        pltpu.async_copy(tab_ref.at[idxv], rows, sem).wait()
        pltpu.sync_copy(rows, out_ref.at[pl.ds(off, 128)])
        return carry

    jax.lax.fori_loop(0, per // 128, body, 0)


def _sc_rowgather(idx, table):
    nrow = idx.shape[0]
    dcol = table.shape[1]
    mesh = plsc.VectorSubcoreMesh(core_axis_name="c", subcore_axis_name="s")
    kfn = pl.kernel(
        functools.partial(_sc_rowgather_kernel, nrow, dcol),
        mesh=mesh,
        out_type=jax.ShapeDtypeStruct((nrow, dcol), jnp.float32),
        scratch_types=[
            pltpu.VMEM((128,), jnp.int32),
            pltpu.VMEM((128, dcol), jnp.float32),
            pltpu.SemaphoreType.DMA,
        ],
        compiler_params=pltpu.CompilerParams(needs_layout_passes=False),
    )
    return kfn(idx, table)


def _nbr_from_dest(dest):
    Bb, m, n = dest.shape
    ci = jnp.broadcast_to(jnp.arange(n, dtype=jnp.int32), (Bb, m, n))
    d = jnp.where(dest >= 0, dest, K)
    nbr = jnp.zeros((Bb, m, K), jnp.int32)
    bi = jnp.arange(Bb, dtype=jnp.int32)[:, None, None]
    qi = jnp.arange(m, dtype=jnp.int32)[None, :, None]
    return nbr.at[bi, qi, d].set(ci, mode="drop")


def _radius(pos_b, qpos, r):
    d2 = jnp.sum((qpos[:, :, None, :] - pos_b[:, None, :, :]) ** 2, axis=-1)
    negv, nbr = jax.lax.top_k(-d2, K)
    mask = (-negv) <= r * r + 1e-12
    return nbr, mask


def _gather(xb, nbr):
    Bb, m, k = nbr.shape
    out = jnp.take_along_axis(xb, nbr.reshape(Bb, m * k)[:, :, None], axis=1)
    return out.reshape(Bb, m, k, xb.shape[-1])


def _take(xb, idx):
    return jnp.take_along_axis(xb, idx[:, :, None], axis=1)


def _mlp_bn(x, layers, mask=None):
    for lyr in layers[:-1]:
        W, b, g, be = lyr
        x = x @ W + b
        if mask is None:
            mu = jnp.mean(x, axis=0)
            var = jnp.mean((x - mu) ** 2, axis=0)
        else:
            w = mask / jnp.maximum(jnp.sum(mask), 1.0)
            mu = jnp.sum(w[:, None] * x, axis=0)
            var = jnp.sum(w[:, None] * (x - mu) ** 2, axis=0)
        x = (x - mu) / jnp.sqrt(var + 1e-5) * g + be
        x = jax.nn.relu(x)
    W, b = layers[-1]
    return x @ W + b


def _point_conv(x_b, pos_b, qpos, nbr, mask, layers):
    h = _gather(pos_b, nbr) - qpos[:, :, None, :]
    if x_b is not None:
        h = jnp.concatenate([_gather(x_b, nbr), h], axis=-1)
    Bb, m, k, F = h.shape
    out = _mlp_bn(h.reshape(Bb * m * k, F), layers, mask.reshape(Bb * m * k).astype(jnp.float32))
    out = out.reshape(Bb, m, k, -1)
    out = jnp.where(mask[..., None], out, -jnp.inf)
    return jnp.max(out, axis=2)


def _tail_kernel(h_ref, w1, b1, g1, be1, w2, b2, g2, be2, w3, b3,
                 pw1, pb1, pw2, pb2, pw3, pb3,
                 vw1, vb1, vw2, vb2, vw3, vb3,
                 probs_ref, value_ref):
    x = h_ref[...]
    x = jnp.dot(x, w1[...], preferred_element_type=jnp.float32) + b1[...]
    mu = jnp.mean(x, axis=0)
    var = jnp.mean((x - mu) ** 2, axis=0)
    x = (x - mu) / jnp.sqrt(var + 1e-5) * g1[...] + be1[...]
    x = jax.nn.relu(x)
    x = jnp.dot(x, w2[...], preferred_element_type=jnp.float32) + b2[...]
    mu = jnp.mean(x, axis=0)
    var = jnp.mean((x - mu) ** 2, axis=0)
    x = (x - mu) / jnp.sqrt(var + 1e-5) * g2[...] + be2[...]
    x = jax.nn.relu(x)
    x = jnp.dot(x, w3[...], preferred_element_type=jnp.float32) + b3[...]
    g = jnp.max(x.reshape(B, M2, x.shape[-1]), axis=1)
    p = jax.nn.relu(jnp.dot(g, pw1[...], preferred_element_type=jnp.float32) + pb1[...])
    p = jax.nn.relu(jnp.dot(p, pw2[...], preferred_element_type=jnp.float32) + pb2[...])
    logits = jnp.dot(p, pw3[...], preferred_element_type=jnp.float32) + pb3[...]
    probs_ref[...] = jax.nn.softmax(logits, axis=-1)
    v = jax.nn.relu(jnp.dot(g, vw1[...], preferred_element_type=jnp.float32) + vb1[...])
    v = jax.nn.relu(jnp.dot(v, vw2[...], preferred_element_type=jnp.float32) + vb2[...])
    value_ref[...] = jnp.dot(v, vw3[...], preferred_element_type=jnp.float32) + vb3[...]


def _dense_tail(h, params):
    (w1, b1, g1, be1), (w2, b2, g2, be2), (w3, b3) = params["nn3"]
    (pw1, pb1), (pw2, pb2), (pw3, pb3) = params["pi"]
    (vw1, vb1), (vw2, vb2), (vw3, vb3) = params["value"]
    probs, value = pl.pallas_call(
        _tail_kernel,
        out_shape=(
            jax.ShapeDtypeStruct((B, N_ACTIONS), jnp.float32),
            jax.ShapeDtypeStruct((B, 1), jnp.float32),
        ),
    )(h, w1, b1, g1, be1, w2, b2, g2, be2, w3, b3,
      pw1, pb1, pw2, pb2, pw3, pb3,
      vw1, vb1, vw2, vb2, vw3, vb3)
    return probs, value[:, 0]


def kernel(pos, ptr, params):
    Bb = ptr.shape[0] - 1
    n = pos.shape[0] // Bb
    pos_b = pos.reshape(Bb, n, 3)
    pos_b = jax.lax.stop_gradient(pos_b)
    xs = pos_b[:, :, 0]
    ys = pos_b[:, :, 1]
    zs = pos_b[:, :, 2]
    (q1, p1x, p1y, p1z, q2, p2x, p2y, p2z) = _fps_pallas(xs, ys, zs)
    pos1 = jnp.stack([p1x, p1y, p1z], axis=-1)
    pos2 = jnp.stack([p2x, p2y, p2z], axis=-1)
    dest1, mask1f = _sel_pallas(p1x.T, p1y.T, p1z.T, xs, ys, zs, R1)
    dest2, mask2f = _sel_pallas(p2x.T, p2y.T, p2z.T, p1x, p1y, p1z, R2)

    h1 = _sc_compact(dest1.reshape(Bb * M1, NPER), xs, ys, zs,
                     p1x, p1y, p1z, False)
    x1 = _pointconv_pallas(h1.reshape(Bb * M1 * K, 3), None,
                           mask1f.reshape(Bb * M1, K), params["nn1"])
    x1pad = jnp.concatenate([x1, jnp.zeros((1, x1.shape[1]), jnp.float32)], axis=0)
    h2, nbr2g = _sc_compact(dest2.reshape(Bb * M2, M1), p1x, p1y, p1z,
                            p2x, p2y, p2z, True)
    xg = _sc_rowgather(nbr2g, x1pad)
    x2 = _pointconv_pallas(h2.reshape(Bb * M2 * K, 3), xg,
                           mask2f.reshape(Bb * M2, K), params["nn2"])
    h = jnp.concatenate([x2.reshape(Bb, M2, -1), pos2], axis=-1).reshape(Bb * M2, -1)
    return _dense_tail(h, params)
